# baseline (speedup 1.0000x reference)
import jax
import jax.numpy as jnp
from jax import lax
from jax.experimental import pallas as pl
from jax.experimental.pallas import tpu as pltpu
from jax.experimental.pallas import tpu_sc as plsc

E = 8
TOP_K = 2
D = 1024
H = 4096
B = 2
S = 2048
N = B * S
NK = N * TOP_K
TILE = 256
P = NK + E * TILE
T = P // TILE

_NC = 2
_NS = 16
_NW = _NC * _NS

_GATE_TILE = 512


def _gate_body(x_ref, wg_ref, e_ref, w_ref):
    x = x_ref[...]
    wg = wg_ref[...]
    logits = lax.dot_general(x, wg, (((1,), (1,)), ((), ())),
                             preferred_element_type=jnp.float32)
    iota = lax.broadcasted_iota(jnp.int32, logits.shape, 1)
    m1 = jnp.max(logits, axis=1, keepdims=True)
    i1 = jnp.min(jnp.where(logits == m1, iota, E), axis=1, keepdims=True)
    l2 = jnp.where(iota == i1, -jnp.inf, logits)
    m2 = jnp.max(l2, axis=1, keepdims=True)
    i2 = jnp.min(jnp.where(l2 == m2, iota, E), axis=1, keepdims=True)
    e_ref[...] = jnp.concatenate([i1, i2], axis=1)
    t = jnp.exp(m2 - m1)
    w1 = 1.0 / (1.0 + t)
    w_ref[...] = jnp.concatenate([w1, w1 * t], axis=1)


def _ffn1_body(te_ref, xs_ref, w1_ref, b1_ref, hs_ref):
    del te_ref
    h = lax.dot_general(xs_ref[...], w1_ref[0], (((1,), (1,)), ((), ())),
                        preferred_element_type=jnp.float32)
    hs_ref[...] = jnp.maximum(h + b1_ref[0], 0.0).astype(jnp.bfloat16)


def _ffn2_body(te_ref, hs_ref, w2_ref, b2_ref, ys_ref):
    del te_ref
    y = lax.dot_general(hs_ref[...], w2_ref[0], (((1,), (1,)), ((), ())),
                        preferred_element_type=jnp.float32)
    ys_ref[...] = y + b2_ref[0]


_DCHUNK = 32


def _dispatch_body(x_hbm, pe2_hbm, po2_hbm, xs_hbm,
                   ie_v, io_v, rows_v, ls0, ls1, sa0, sb0, sa1, sb1):
    wid = lax.axis_index("s") * _NC + lax.axis_index("c")
    per_w = N // _NW
    nch = per_w // _DCHUNK
    base = wid * per_w
    wrow = wid * nch
    pltpu.sync_copy(pe2_hbm.at[pl.ds(wrow, nch)], ie_v)
    pltpu.sync_copy(po2_hbm.at[pl.ds(wrow, nch)], io_v)
    ls = (ls0, ls1)
    sa = (sa0, sa1)
    sb = (sb0, sb1)
    loads = [None] * nch
    scats = [None] * nch
    loads[0] = pltpu.async_copy(x_hbm.at[pl.ds(base, _DCHUNK)], rows_v.at[0], ls[0])
    for c in range(nch):
        b = c % 2
        loads[c].wait()
        if c + 1 < nch:
            if c >= 1:
                scats[c - 1][0].wait()
                scats[c - 1][1].wait()
            loads[c + 1] = pltpu.async_copy(
                x_hbm.at[pl.ds(base + (c + 1) * _DCHUNK, _DCHUNK)],
                rows_v.at[(c + 1) % 2], ls[(c + 1) % 2])
        scats[c] = (
            pltpu.async_copy(rows_v.at[b], xs_hbm.at[ie_v.at[c]], sa[b]),
            pltpu.async_copy(rows_v.at[b], xs_hbm.at[io_v.at[c]], sb[b]),
        )
    for c in (nch - 2, nch - 1):
        scats[c][0].wait()
        scats[c][1].wait()


_CCHUNK = 16


def _combine_body(ys_hbm, pi2_hbm, we_hbm, wo_hbm, out_hbm,
                  pi_v, we_v, wo_v, g_v, o_v, gs0, gs1, os0, os1):
    wid = lax.axis_index("s") * _NC + lax.axis_index("c")
    per_w = N // _NW
    nch = per_w // _CCHUNK
    base = wid * per_w
    wrow = wid * nch
    pltpu.sync_copy(pi2_hbm.at[pl.ds(wrow, nch)], pi_v)
    pltpu.sync_copy(we_hbm.at[pl.ds(base, per_w)], we_v)
    pltpu.sync_copy(wo_hbm.at[pl.ds(base, per_w)], wo_v)
    gs = (gs0, gs1)
    osem = (os0, os1)
    gathers = [None] * nch
    owrites = [None] * nch
    gathers[0] = pltpu.async_copy(ys_hbm.at[pi_v.at[0]], g_v.at[0], gs[0])
    for c in range(nch):
        b = c % 2
        gathers[c].wait()
        if c + 1 < nch:
            gathers[c + 1] = pltpu.async_copy(
                ys_hbm.at[pi_v.at[c + 1]], g_v.at[(c + 1) % 2], gs[(c + 1) % 2])
        if c >= 1:
            owrites[c - 1].wait()
        gb = g_v.at[b]
        ob = o_v

        def _row(i, _):
            wa = we_v[c * _CCHUNK + i, :]
            wb = wo_v[c * _CCHUNK + i, :]

            def _vec(u, _):
                sl = pl.ds(u * 16, 16)
                ob[i, sl] = wa * gb[2 * i, sl] + wb * gb[2 * i + 1, sl]
                return 0

            lax.fori_loop(0, D // 16, _vec, 0, unroll=4)
            return 0

        lax.fori_loop(0, _CCHUNK, _row, 0)
        owrites[c] = pltpu.async_copy(
            ob, out_hbm.at[pl.ds(base + c * _CCHUNK, _CCHUNK)], osem[b])
    owrites[nch - 1].wait()


def kernel(x, Wg, W1, b1, W2, b2):
    x2 = x.reshape(N, D)

    experts, weights = pl.pallas_call(
        _gate_body,
        grid=(N // _GATE_TILE,),
        in_specs=[
            pl.BlockSpec((_GATE_TILE, D), lambda i: (i, 0)),
            pl.BlockSpec((E, D), lambda i: (0, 0)),
        ],
        out_specs=[
            pl.BlockSpec((_GATE_TILE, TOP_K), lambda i: (i, 0)),
            pl.BlockSpec((_GATE_TILE, TOP_K), lambda i: (i, 0)),
        ],
        out_shape=[
            jax.ShapeDtypeStruct((N, TOP_K), jnp.int32),
            jax.ShapeDtypeStruct((N, TOP_K), jnp.float32),
        ],
    )(x2, Wg)

    ef = experts.reshape(NK)
    wf = weights.reshape(NK)
    oh = (ef[:, None] == jnp.arange(E, dtype=jnp.int32)[None, :]).astype(jnp.int32)
    cnt = jnp.sum(oh, axis=0)
    gsz = ((cnt + TILE - 1) // TILE) * TILE
    ends = jnp.cumsum(gsz)
    starts = ends - gsz
    rank = jnp.sum((jnp.cumsum(oh, axis=0) - oh) * oh, axis=1)
    pos = (jnp.sum(oh * starts[None, :], axis=1) + rank).astype(jnp.int32)
    tile_expert = jnp.minimum(
        jnp.sum(ends[None, :] <= (jnp.arange(T, dtype=jnp.int32) * TILE)[:, None],
                axis=1), E - 1).astype(jnp.int32)
    pos2 = pos.reshape(N, TOP_K)
    pe = pos2[:, 0]
    po = pos2[:, 1]
    we = jnp.broadcast_to(weights[:, 0:1], (N, 16))
    wo = jnp.broadcast_to(weights[:, 1:2], (N, 16))

    nchd = (N // _NW) // _DCHUNK
    mesh = plsc.VectorSubcoreMesh(core_axis_name="c", subcore_axis_name="s")
    xs = pl.kernel(
        _dispatch_body,
        out_type=jax.ShapeDtypeStruct((P, D), jnp.float32),
        mesh=mesh,
        scratch_types=[
            pltpu.VMEM((nchd, _DCHUNK), jnp.int32),
            pltpu.VMEM((nchd, _DCHUNK), jnp.int32),
            pltpu.VMEM((2, _DCHUNK, D), jnp.float32),
            pltpu.SemaphoreType.DMA,
            pltpu.SemaphoreType.DMA,
            pltpu.SemaphoreType.DMA,
            pltpu.SemaphoreType.DMA,
            pltpu.SemaphoreType.DMA,
            pltpu.SemaphoreType.DMA,
        ],
    )(x2, pe.reshape(N // _DCHUNK, _DCHUNK), po.reshape(N // _DCHUNK, _DCHUNK))

    hs = pl.pallas_call(
        _ffn1_body,
        grid_spec=pltpu.PrefetchScalarGridSpec(
            num_scalar_prefetch=1,
            grid=(T,),
            in_specs=[
                pl.BlockSpec((TILE, D), lambda t, te: (t, 0)),
                pl.BlockSpec((1, H, D), lambda t, te: (te[t], 0, 0)),
                pl.BlockSpec((1, 1, H), lambda t, te: (te[t], 0, 0)),
            ],
            out_specs=pl.BlockSpec((TILE, H), lambda t, te: (t, 0)),
        ),
        out_shape=jax.ShapeDtypeStruct((P, H), jnp.bfloat16),
        compiler_params=pltpu.CompilerParams(vmem_limit_bytes=60 * 1024 * 1024),
    )(tile_expert, xs, W1, b1.reshape(E, 1, H))
    ys = pl.pallas_call(
        _ffn2_body,
        grid_spec=pltpu.PrefetchScalarGridSpec(
            num_scalar_prefetch=1,
            grid=(T,),
            in_specs=[
                pl.BlockSpec((TILE, H), lambda t, te: (t, 0)),
                pl.BlockSpec((1, D, H), lambda t, te: (te[t], 0, 0)),
                pl.BlockSpec((1, 1, D), lambda t, te: (te[t], 0, 0)),
            ],
            out_specs=pl.BlockSpec((TILE, D), lambda t, te: (t, 0)),
        ),
        out_shape=jax.ShapeDtypeStruct((P, D), jnp.float32),
        compiler_params=pltpu.CompilerParams(vmem_limit_bytes=60 * 1024 * 1024),
    )(tile_expert, hs, W2, b2.reshape(E, 1, D))

    nchc = (N // _NW) // _CCHUNK
    out = pl.kernel(
        _combine_body,
        out_type=jax.ShapeDtypeStruct((N, D), jnp.float32),
        mesh=mesh,
        scratch_types=[
            pltpu.VMEM((nchc, 2 * _CCHUNK), jnp.int32),
            pltpu.VMEM((N // _NW, 16), jnp.float32),
            pltpu.VMEM((N // _NW, 16), jnp.float32),
            pltpu.VMEM((2, 2 * _CCHUNK, D), jnp.float32),
            pltpu.VMEM((_CCHUNK, D), jnp.float32),
            pltpu.SemaphoreType.DMA,
            pltpu.SemaphoreType.DMA,
            pltpu.SemaphoreType.DMA,
            pltpu.SemaphoreType.DMA,
        ],
    )(ys, pos.reshape(NK // (2 * _CCHUNK), 2 * _CCHUNK), we, wo)

    return (out.reshape(B, S, D), experts.reshape(B, S, TOP_K))


# speedup vs baseline: 1.0632x; 1.0632x over previous
import jax
import jax.numpy as jnp
from jax import lax
from jax.experimental import pallas as pl
from jax.experimental.pallas import tpu as pltpu
from jax.experimental.pallas import tpu_sc as plsc

E = 8
TOP_K = 2
D = 1024
H = 4096
B = 2
S = 2048
N = B * S
NK = N * TOP_K
TILE = 256
P = NK + E * TILE
T = P // TILE

_NC = 2
_NS = 16
_NW = _NC * _NS

_GATE_TILE = 512


def _gate_body(x_ref, wg_ref, e_ref, w_ref):
    x = x_ref[...]
    wg = wg_ref[...]
    logits = lax.dot_general(x, wg, (((1,), (1,)), ((), ())),
                             preferred_element_type=jnp.float32)
    iota = lax.broadcasted_iota(jnp.int32, logits.shape, 1)
    m1 = jnp.max(logits, axis=1, keepdims=True)
    i1 = jnp.min(jnp.where(logits == m1, iota, E), axis=1, keepdims=True)
    l2 = jnp.where(iota == i1, -jnp.inf, logits)
    m2 = jnp.max(l2, axis=1, keepdims=True)
    i2 = jnp.min(jnp.where(l2 == m2, iota, E), axis=1, keepdims=True)
    e_ref[...] = jnp.concatenate([i1, i2], axis=1)
    t = jnp.exp(m2 - m1)
    w1 = 1.0 / (1.0 + t)
    w_ref[...] = jnp.concatenate([w1, w1 * t], axis=1)


def _ffn1_body(te_ref, xs_ref, w1_ref, b1_ref, hs_ref):
    del te_ref
    h = lax.dot_general(xs_ref[...], w1_ref[0], (((1,), (1,)), ((), ())),
                        preferred_element_type=jnp.float32)
    hs_ref[...] = jnp.maximum(h + b1_ref[0], 0.0).astype(jnp.bfloat16)


def _ffn2_body(te_ref, hs_ref, w2_ref, b2_ref, ys_ref):
    del te_ref
    y = lax.dot_general(hs_ref[...], w2_ref[0], (((1,), (1,)), ((), ())),
                        preferred_element_type=jnp.float32)
    ys_ref[...] = y + b2_ref[0]


_DCHUNK = 32


def _dispatch_body(x_hbm, pe2_hbm, po2_hbm, xs_hbm,
                   ie_v, io_v, rows_v, ls0, ls1, sa0, sb0, sa1, sb1):
    wid = lax.axis_index("s") * _NC + lax.axis_index("c")
    per_w = N // _NW
    nch = per_w // _DCHUNK
    base = wid * per_w
    wrow = wid * nch
    pltpu.sync_copy(pe2_hbm.at[pl.ds(wrow, nch)], ie_v)
    pltpu.sync_copy(po2_hbm.at[pl.ds(wrow, nch)], io_v)
    ls = (ls0, ls1)
    sa = (sa0, sa1)
    sb = (sb0, sb1)
    loads = [None] * nch
    scats = [None] * nch
    loads[0] = pltpu.async_copy(x_hbm.at[pl.ds(base, _DCHUNK)], rows_v.at[0], ls[0])
    for c in range(nch):
        b = c % 2
        loads[c].wait()
        if c + 1 < nch:
            if c >= 1:
                scats[c - 1][0].wait()
                scats[c - 1][1].wait()
            loads[c + 1] = pltpu.async_copy(
                x_hbm.at[pl.ds(base + (c + 1) * _DCHUNK, _DCHUNK)],
                rows_v.at[(c + 1) % 2], ls[(c + 1) % 2])
        scats[c] = (
            pltpu.async_copy(rows_v.at[b], xs_hbm.at[ie_v.at[c]], sa[b]),
            pltpu.async_copy(rows_v.at[b], xs_hbm.at[io_v.at[c]], sb[b]),
        )
    for c in (nch - 2, nch - 1):
        scats[c][0].wait()
        scats[c][1].wait()


_CCHUNK = 32


def _combine_body(ys_hbm, pe_hbm, po_hbm, we_hbm, wo_hbm, out_hbm,
                  ie_v, io_v, we_v, wo_v, a_v, b_v, o_v, sem):
    wid = lax.axis_index("s") * _NC + lax.axis_index("c")
    per_w = N // _NW
    base = wid * per_w
    for ch in range(per_w // _CCHUNK):
        tok0 = base + ch * _CCHUNK
        pltpu.sync_copy(pe_hbm.at[pl.ds(tok0, _CCHUNK)], ie_v)
        pltpu.sync_copy(po_hbm.at[pl.ds(tok0, _CCHUNK)], io_v)
        pltpu.sync_copy(we_hbm.at[pl.ds(tok0, _CCHUNK)], we_v)
        pltpu.sync_copy(wo_hbm.at[pl.ds(tok0, _CCHUNK)], wo_v)
        cpa = pltpu.async_copy(ys_hbm.at[ie_v], a_v, sem)
        cpb = pltpu.async_copy(ys_hbm.at[io_v], b_v, sem)
        cpa.wait()
        cpb.wait()

        def _row(i, _):
            wa = we_v[i, :]
            wb = wo_v[i, :]

            def _vec(u, _):
                sl = pl.ds(u * 16, 16)
                o_v[i, sl] = wa * a_v[i, sl] + wb * b_v[i, sl]
                return 0

            lax.fori_loop(0, D // 16, _vec, 0, unroll=4)
            return 0

        lax.fori_loop(0, _CCHUNK, _row, 0)
        pltpu.sync_copy(o_v, out_hbm.at[pl.ds(tok0, _CCHUNK)])


def kernel(x, Wg, W1, b1, W2, b2):
    x2 = x.reshape(N, D)

    experts, weights = pl.pallas_call(
        _gate_body,
        grid=(N // _GATE_TILE,),
        in_specs=[
            pl.BlockSpec((_GATE_TILE, D), lambda i: (i, 0)),
            pl.BlockSpec((E, D), lambda i: (0, 0)),
        ],
        out_specs=[
            pl.BlockSpec((_GATE_TILE, TOP_K), lambda i: (i, 0)),
            pl.BlockSpec((_GATE_TILE, TOP_K), lambda i: (i, 0)),
        ],
        out_shape=[
            jax.ShapeDtypeStruct((N, TOP_K), jnp.int32),
            jax.ShapeDtypeStruct((N, TOP_K), jnp.float32),
        ],
    )(x2, Wg)

    ef = experts.reshape(NK)
    wf = weights.reshape(NK)
    oh = (ef[:, None] == jnp.arange(E, dtype=jnp.int32)[None, :]).astype(jnp.int32)
    cnt = jnp.sum(oh, axis=0)
    gsz = ((cnt + TILE - 1) // TILE) * TILE
    ends = jnp.cumsum(gsz)
    starts = ends - gsz
    rank = jnp.sum((jnp.cumsum(oh, axis=0) - oh) * oh, axis=1)
    pos = (jnp.sum(oh * starts[None, :], axis=1) + rank).astype(jnp.int32)
    tile_expert = jnp.minimum(
        jnp.sum(ends[None, :] <= (jnp.arange(T, dtype=jnp.int32) * TILE)[:, None],
                axis=1), E - 1).astype(jnp.int32)
    pos2 = pos.reshape(N, TOP_K)
    pe = pos2[:, 0]
    po = pos2[:, 1]
    we = jnp.broadcast_to(weights[:, 0:1], (N, 16))
    wo = jnp.broadcast_to(weights[:, 1:2], (N, 16))

    nchd = (N // _NW) // _DCHUNK
    mesh = plsc.VectorSubcoreMesh(core_axis_name="c", subcore_axis_name="s")
    xs = pl.kernel(
        _dispatch_body,
        out_type=jax.ShapeDtypeStruct((P, D), jnp.float32),
        mesh=mesh,
        scratch_types=[
            pltpu.VMEM((nchd, _DCHUNK), jnp.int32),
            pltpu.VMEM((nchd, _DCHUNK), jnp.int32),
            pltpu.VMEM((2, _DCHUNK, D), jnp.float32),
            pltpu.SemaphoreType.DMA,
            pltpu.SemaphoreType.DMA,
            pltpu.SemaphoreType.DMA,
            pltpu.SemaphoreType.DMA,
            pltpu.SemaphoreType.DMA,
            pltpu.SemaphoreType.DMA,
        ],
    )(x2, pe.reshape(N // _DCHUNK, _DCHUNK), po.reshape(N // _DCHUNK, _DCHUNK))

    hs = pl.pallas_call(
        _ffn1_body,
        grid_spec=pltpu.PrefetchScalarGridSpec(
            num_scalar_prefetch=1,
            grid=(T,),
            in_specs=[
                pl.BlockSpec((TILE, D), lambda t, te: (t, 0)),
                pl.BlockSpec((1, H, D), lambda t, te: (te[t], 0, 0)),
                pl.BlockSpec((1, 1, H), lambda t, te: (te[t], 0, 0)),
            ],
            out_specs=pl.BlockSpec((TILE, H), lambda t, te: (t, 0)),
        ),
        out_shape=jax.ShapeDtypeStruct((P, H), jnp.bfloat16),
        compiler_params=pltpu.CompilerParams(vmem_limit_bytes=60 * 1024 * 1024),
    )(tile_expert, xs, W1, b1.reshape(E, 1, H))
    ys = pl.pallas_call(
        _ffn2_body,
        grid_spec=pltpu.PrefetchScalarGridSpec(
            num_scalar_prefetch=1,
            grid=(T,),
            in_specs=[
                pl.BlockSpec((TILE, H), lambda t, te: (t, 0)),
                pl.BlockSpec((1, D, H), lambda t, te: (te[t], 0, 0)),
                pl.BlockSpec((1, 1, D), lambda t, te: (te[t], 0, 0)),
            ],
            out_specs=pl.BlockSpec((TILE, D), lambda t, te: (t, 0)),
        ),
        out_shape=jax.ShapeDtypeStruct((P, D), jnp.float32),
        compiler_params=pltpu.CompilerParams(vmem_limit_bytes=60 * 1024 * 1024),
    )(tile_expert, hs, W2, b2.reshape(E, 1, D))

    out = pl.kernel(
        _combine_body,
        out_type=jax.ShapeDtypeStruct((N, D), jnp.float32),
        mesh=mesh,
        scratch_types=[
            pltpu.VMEM((_CCHUNK,), jnp.int32),
            pltpu.VMEM((_CCHUNK,), jnp.int32),
            pltpu.VMEM((_CCHUNK, 16), jnp.float32),
            pltpu.VMEM((_CCHUNK, 16), jnp.float32),
            pltpu.VMEM((_CCHUNK, D), jnp.float32),
            pltpu.VMEM((_CCHUNK, D), jnp.float32),
            pltpu.VMEM((_CCHUNK, D), jnp.float32),
            pltpu.SemaphoreType.DMA,
        ],
    )(ys, pe, po, we, wo)

    return (out.reshape(B, S, D), experts.reshape(B, S, TOP_K))
